# baseline (speedup 1.0000x reference)
import functools

import jax
import jax.numpy as jnp
from jax import lax
from jax.experimental import pallas as pl
from jax.experimental.pallas import tpu as pltpu
from jax.experimental.pallas import tpu_sc as plsc

B = 16384
D = 128
NC = 2
NS = 16
NW = NC * NS
BPW = B // NW
CH = 128
NCH = BPW // CH
GROUPS = CH // 16


def _mirt_body(uids_hbm, iids_hbm, emb_hbm, disc_hbm, diff_hbm, out_hbm,
               uid_v, iid_v, theta_v, a_v, b_v, out_v, sem):
    wid = lax.axis_index("s") * NC + lax.axis_index("c")

    pltpu.sync_copy(uids_hbm.at[wid], uid_v)
    pltpu.sync_copy(iids_hbm.at[wid], iid_v)

    iota = lax.iota(jnp.int32, 16)

    for c in range(NCH):
        cp_t = pltpu.async_copy(emb_hbm.at[uid_v.at[c]], theta_v, sem)
        cp_a = pltpu.async_copy(disc_hbm.at[iid_v.at[c]], a_v, sem)
        cp_b = pltpu.async_copy(diff_hbm.at[iid_v.at[c]], b_v, sem)
        cp_t.wait()
        cp_a.wait()
        cp_b.wait()

        def group_body(g, _, c=c):
            acc = jnp.zeros((16,), jnp.float32)
            rows = g * 16 + iota
            cols = jnp.zeros((16,), jnp.int32)
            for _d in range(D):
                tv = plsc.load_gather(theta_v, [rows, cols])
                av = plsc.load_gather(a_v, [rows, cols])
                acc = acc + tv * av
                cols = cols + 1
            b16 = b_v[pl.ds(g * 16, 16)]
            x = acc - b16
            out_v[c, pl.ds(g * 16, 16)] = 1.0 / (1.0 + jnp.exp(-x))
            return 0

        lax.fori_loop(0, GROUPS, group_body, 0)

    pltpu.sync_copy(out_v, out_hbm.at[wid])


@jax.jit
def _mirt_sc(uids, iids, users_emb, item_discrimination, diff):
    mesh = plsc.VectorSubcoreMesh(core_axis_name="c", subcore_axis_name="s")
    run = pl.kernel(
        _mirt_body,
        out_type=jax.ShapeDtypeStruct((NW, NCH, CH), jnp.float32),
        mesh=mesh,
        scratch_types=[
            pltpu.VMEM((NCH, CH), jnp.int32),
            pltpu.VMEM((NCH, CH), jnp.int32),
            pltpu.VMEM((CH, D), jnp.float32),
            pltpu.VMEM((CH, D), jnp.float32),
            pltpu.VMEM((CH,), jnp.float32),
            pltpu.VMEM((NCH, CH), jnp.float32),
            pltpu.SemaphoreType.DMA,
        ],
        compiler_params=pltpu.CompilerParams(needs_layout_passes=False),
    )
    return run(uids, iids, users_emb, item_discrimination, diff)


def kernel(user_ids, item_ids, concept_ids, users_emb, item_discrimination,
           item_difficulty):
    del concept_ids
    uids = user_ids.astype(jnp.int32).reshape(NW, NCH, CH)
    iids = item_ids.astype(jnp.int32).reshape(NW, NCH, CH)
    diff = item_difficulty.reshape(-1)
    out = _mirt_sc(uids, iids, users_emb, item_discrimination, diff)
    return out.reshape(B)


# speedup vs baseline: 2.0736x; 2.0736x over previous
import jax
import jax.numpy as jnp
from jax import lax
from jax.experimental import pallas as pl
from jax.experimental.pallas import tpu as pltpu
from jax.experimental.pallas import tpu_sc as plsc

B = 16384
D = 128
NC = 2
NS = 16
NW = NC * NS
BPW = B // NW
CH = 128
NCH = BPW // CH
GROUPS = CH // 16


def _mirt_body(uids_hbm, iids_hbm, emb_hbm, disc_hbm, diff_hbm, out_hbm,
               uid_v, iid_v, theta_v, a_v, b_v, out_v, sem0, sem1):
    wid = lax.axis_index("s") * NC + lax.axis_index("c")

    pltpu.sync_copy(uids_hbm.at[wid], uid_v)
    pltpu.sync_copy(iids_hbm.at[wid], iid_v)

    iota = lax.iota(jnp.int32, 16)
    sems = (sem0, sem1)

    def start(c):
        p = c % 2
        return (
            pltpu.async_copy(emb_hbm.at[uid_v.at[c]], theta_v.at[p], sems[p]),
            pltpu.async_copy(disc_hbm.at[iid_v.at[c]], a_v.at[p], sems[p]),
            pltpu.async_copy(diff_hbm.at[iid_v.at[c]], b_v.at[p], sems[p]),
        )

    pending = start(0)
    for c in range(NCH):
        p = c % 2
        for cp in pending:
            cp.wait()
        if c + 1 < NCH:
            pending = start(c + 1)

        def group_body(g, _, p=p, c=c):
            out16 = jnp.zeros((16,), jnp.float32)
            for e16 in range(16):
                e = g * 16 + e16
                acc = None
                for j in range(D // 16):
                    tv = theta_v[p, e, pl.ds(j * 16, 16)]
                    av = a_v[p, e, pl.ds(j * 16, 16)]
                    prod = tv * av
                    acc = prod if acc is None else acc + prod
                s = jnp.sum(acc)
                out16 = jnp.where(iota == e16, s, out16)
            b16 = b_v[p, pl.ds(g * 16, 16)]
            x = out16 - b16
            out_v[c, pl.ds(g * 16, 16)] = 1.0 / (1.0 + jnp.exp(-x))
            return 0

        lax.fori_loop(0, GROUPS, group_body, 0)

    pltpu.sync_copy(out_v, out_hbm.at[wid])


@jax.jit
def _mirt_sc(uids, iids, users_emb, item_discrimination, diff):
    mesh = plsc.VectorSubcoreMesh(core_axis_name="c", subcore_axis_name="s")
    run = pl.kernel(
        _mirt_body,
        out_type=jax.ShapeDtypeStruct((NW, NCH, CH), jnp.float32),
        mesh=mesh,
        scratch_types=[
            pltpu.VMEM((NCH, CH), jnp.int32),
            pltpu.VMEM((NCH, CH), jnp.int32),
            pltpu.VMEM((2, CH, D), jnp.float32),
            pltpu.VMEM((2, CH, D), jnp.float32),
            pltpu.VMEM((2, CH), jnp.float32),
            pltpu.VMEM((NCH, CH), jnp.float32),
            pltpu.SemaphoreType.DMA,
            pltpu.SemaphoreType.DMA,
        ],
        compiler_params=pltpu.CompilerParams(needs_layout_passes=False),
    )
    return run(uids, iids, users_emb, item_discrimination, diff)


def kernel(user_ids, item_ids, concept_ids, users_emb, item_discrimination,
           item_difficulty):
    del concept_ids
    uids = user_ids.astype(jnp.int32).reshape(NW, NCH, CH)
    iids = item_ids.astype(jnp.int32).reshape(NW, NCH, CH)
    diff = item_difficulty.reshape(-1)
    out = _mirt_sc(uids, iids, users_emb, item_discrimination, diff)
    return out.reshape(B)


# speedup vs baseline: 2.9188x; 1.4076x over previous
import jax
import jax.numpy as jnp
from jax import lax
from jax.experimental import pallas as pl
from jax.experimental.pallas import tpu as pltpu
from jax.experimental.pallas import tpu_sc as plsc

B = 16384
D = 128
NC = 2
NS = 16
NW = NC * NS
BPW = B // NW
CH = 128
NCH = BPW // CH
GROUPS = CH // 16


def _mirt_body(uids_hbm, iids_hbm, emb_hbm, disc_hbm, diff_hbm, out_hbm,
               uid_v, iid_v, theta_v, a_v, b_v, out_v, pscr_v, sem0, sem1):
    wid = lax.axis_index("s") * NC + lax.axis_index("c")

    pltpu.sync_copy(uids_hbm.at[wid], uid_v)
    pltpu.sync_copy(iids_hbm.at[wid], iid_v)

    iota = lax.iota(jnp.int32, 16)
    sems = (sem0, sem1)

    def start(c):
        p = c % 2
        return (
            pltpu.async_copy(emb_hbm.at[uid_v.at[c]], theta_v.at[p], sems[p]),
            pltpu.async_copy(disc_hbm.at[iid_v.at[c]], a_v.at[p], sems[p]),
            pltpu.async_copy(diff_hbm.at[iid_v.at[c]], b_v.at[p], sems[p]),
        )

    pending = start(0)
    for c in range(NCH):
        p = c % 2
        for cp in pending:
            cp.wait()
        if c + 1 < NCH:
            pending = start(c + 1)

        def group_body(g, _, p=p, c=c):
            for e16 in range(16):
                e = g * 16 + e16
                acc = None
                for j in range(D // 16):
                    tv = theta_v[p, e, pl.ds(j * 16, 16)]
                    av = a_v[p, e, pl.ds(j * 16, 16)]
                    prod = tv * av
                    acc = prod if acc is None else acc + prod
                pscr_v[pl.ds(e16 * 17, 16)] = acc
            idx = iota * 17
            out16 = None
            for j in range(16):
                col = plsc.load_gather(pscr_v, [idx])
                out16 = col if out16 is None else out16 + col
                idx = idx + 1
            b16 = b_v[p, pl.ds(g * 16, 16)]
            x = out16 - b16
            out_v[c, pl.ds(g * 16, 16)] = 1.0 / (1.0 + jnp.exp(-x))
            return 0

        lax.fori_loop(0, GROUPS, group_body, 0)

    pltpu.sync_copy(out_v, out_hbm.at[wid])


@jax.jit
def _mirt_sc(uids, iids, users_emb, item_discrimination, diff):
    mesh = plsc.VectorSubcoreMesh(core_axis_name="c", subcore_axis_name="s")
    run = pl.kernel(
        _mirt_body,
        out_type=jax.ShapeDtypeStruct((NW, NCH, CH), jnp.float32),
        mesh=mesh,
        scratch_types=[
            pltpu.VMEM((NCH, CH), jnp.int32),
            pltpu.VMEM((NCH, CH), jnp.int32),
            pltpu.VMEM((2, CH, D), jnp.float32),
            pltpu.VMEM((2, CH, D), jnp.float32),
            pltpu.VMEM((2, CH), jnp.float32),
            pltpu.VMEM((NCH, CH), jnp.float32),
            pltpu.VMEM((16 * 17,), jnp.float32),
            pltpu.SemaphoreType.DMA,
            pltpu.SemaphoreType.DMA,
        ],
        compiler_params=pltpu.CompilerParams(needs_layout_passes=False),
    )
    return run(uids, iids, users_emb, item_discrimination, diff)


def kernel(user_ids, item_ids, concept_ids, users_emb, item_discrimination,
           item_difficulty):
    del concept_ids
    uids = user_ids.astype(jnp.int32).reshape(NW, NCH, CH)
    iids = item_ids.astype(jnp.int32).reshape(NW, NCH, CH)
    diff = item_difficulty.reshape(-1)
    out = _mirt_sc(uids, iids, users_emb, item_discrimination, diff)
    return out.reshape(B)
